# baseline (device time: 94416 ns/iter reference)
import jax
import jax.numpy as jnp
from jax import lax
from jax.experimental import pallas as pl
from jax.experimental.pallas import tpu as pltpu

S = 1024
D = 2048
DC = 128
H = 16
G = 4
DH = 128
DR = 32
NSTEP = H // G
SCALE = (DH + DR) ** -0.5
LOG2E = 1.4426950408889634
QSCALE = SCALE * LOG2E

F32 = jnp.float32
BF16 = jnp.bfloat16


def kernel(x, Wdkv, Wuk, Wuv, Wq, Wqr, Wkr, Wo):
    def body(x_ref, wdkv_ref, wuk_ref, wuv_ref, wq_ref, wqr_ref, wkr_ref,
             wo_ref, out_ref,
             c_send, c_recv, wuk_send, wuk_recv, wuv_send, wuv_recv, kr_buf,
             send_sems, recv_sems):
        h = pl.program_id(0)
        my_x = lax.axis_index("x")
        my_y = lax.axis_index("y")
        my_z = lax.axis_index("z")
        peer = (1 - my_x, my_y, my_z)

        CW = G * DH

        def mk_rdma(i, src, dst):
            return pltpu.make_async_remote_copy(
                src_ref=src, dst_ref=dst,
                send_sem=send_sems.at[i], recv_sem=recv_sems.at[i],
                device_id=peer, device_id_type=pl.DeviceIdType.MESH)

        def wuk_rdma(g):
            sl = slice(g * CW, (g + 1) * CW)
            return mk_rdma(1 + g, wuk_send.at[:, sl], wuk_recv.at[:, sl])

        def wuv_rdma(g):
            sl = slice(g * CW, (g + 1) * CW)
            return mk_rdma(1 + NSTEP + g, wuv_send.at[:, sl],
                           wuv_recv.at[:, sl])

        @pl.when(h == 0)
        def _exchange():
            barrier_sem = pltpu.get_barrier_semaphore()
            pl.semaphore_signal(barrier_sem, inc=1, device_id=peer,
                                device_id_type=pl.DeviceIdType.MESH)
            pl.semaphore_wait(barrier_sem, 1)

            c_send[...] = jnp.dot(
                x_ref[0].astype(BF16), wdkv_ref[...].astype(BF16),
                preferred_element_type=F32).astype(BF16)
            mk_rdma(0, c_send, c_recv).start()

            c0 = slice(0, CW)
            wuk_send[:, c0] = wuk_ref[:, c0].astype(BF16)
            wuv_send[:, c0] = wuv_ref[:, c0].astype(BF16)
            wuk_rdma(0).start()
            wuv_rdma(0).start()
            rest = slice(CW, D)
            wuk_send[:, rest] = wuk_ref[:, rest].astype(BF16)
            wuv_send[:, rest] = wuv_ref[:, rest].astype(BF16)
            for g in range(1, NSTEP):
                wuk_rdma(g).start()
                wuv_rdma(g).start()

            kr_buf[...] = lax.dot_general(
                x_ref[0], wkr_ref[...], (((1,), (1,)), ((), ())),
                preferred_element_type=F32)

        xb = x_ref[0].astype(BF16)
        q_blk = (jnp.dot(xb, wq_ref[...].astype(BF16),
                         preferred_element_type=F32)
                 * QSCALE).astype(BF16)
        qr_blk = (jnp.dot(xb, wqr_ref[...].astype(BF16),
                          preferred_element_type=F32)
                  * QSCALE).astype(BF16)

        @pl.when(h == 0)
        def _wait_c():
            mk_rdma(0, c_send, c_recv).wait_recv()

        for g in range(NSTEP):
            @pl.when(h == g)
            def _wait_chunks(g=g):
                wuk_rdma(g).wait_recv()

        @pl.when(h == NSTEP - 1)
        def _wait_sends():
            mk_rdma(0, c_send, c_recv).wait_send()
            for g in range(NSTEP):
                wuk_rdma(g).wait_send()
                wuv_rdma(g).wait_send()

        c_m = c_send[...]
        c_p = c_recv[...]
        kr = kr_buf[...].astype(BF16)

        blk = pl.ds(h * (G * DH), G * DH)
        k_blk = (jnp.dot(c_m, wuk_send[:, blk], preferred_element_type=F32)
                 + jnp.dot(c_p, wuk_recv[:, blk],
                           preferred_element_type=F32)).astype(BF16)

        ps, rs = [], []
        for j in range(G):
            k_h = k_blk[:, j * DH:(j + 1) * DH]
            q_h = q_blk[:, j * DH:(j + 1) * DH]
            qr_h = qr_blk[:, j * DR:(j + 1) * DR]
            s = lax.dot_general(q_h, k_h, (((1,), (1,)), ((), ())),
                                preferred_element_type=F32)
            s = s + lax.dot_general(qr_h, kr, (((1,), (1,)), ((), ())),
                                    preferred_element_type=F32)
            p = jnp.exp2(s).astype(BF16)
            ps.append(p)
            rs.append(1.0 / jnp.sum(p.astype(F32), axis=1, keepdims=True))

        for g in range(NSTEP):
            @pl.when(h == g)
            def _wait_v_chunk(g=g):
                wuv_rdma(g).wait_recv()

        v_blk = (jnp.dot(c_m, wuv_send[:, blk], preferred_element_type=F32)
                 + jnp.dot(c_p, wuv_recv[:, blk],
                           preferred_element_type=F32)).astype(BF16)

        o_parts = []
        for j in range(G):
            v_h = v_blk[:, j * DH:(j + 1) * DH]
            o_parts.append(
                jnp.dot(ps[j], v_h, preferred_element_type=F32) * rs[j])

        o_blk = jnp.concatenate(o_parts, axis=1).astype(BF16)
        contrib = jnp.dot(o_blk, wo_ref[...].astype(BF16),
                          preferred_element_type=F32)

        @pl.when(h == 0)
        def _init():
            out_ref[0] = contrib

        @pl.when(h != 0)
        def _acc():
            out_ref[0] = out_ref[0] + contrib

    out = pl.pallas_call(
        body,
        grid=(H // G,),
        out_shape=jax.ShapeDtypeStruct((1, S, D), F32),
        in_specs=[
            pl.BlockSpec((1, S, D), lambda h: (0, 0, 0)),
            pl.BlockSpec((D, DC), lambda h: (0, 0)),
            pl.BlockSpec((DC, D), lambda h: (0, 0)),
            pl.BlockSpec((DC, D), lambda h: (0, 0)),
            pl.BlockSpec((D, G * DH), lambda h: (0, h)),
            pl.BlockSpec((D, G * DR), lambda h: (0, h)),
            pl.BlockSpec((DR, D), lambda h: (0, 0)),
            pl.BlockSpec((G * DH, D), lambda h: (h, 0)),
        ],
        out_specs=pl.BlockSpec((1, S, D), lambda h: (0, 0, 0)),
        scratch_shapes=[
            pltpu.VMEM((S, DC), BF16),
            pltpu.VMEM((S, DC), BF16),
            pltpu.VMEM((DC, D), BF16),
            pltpu.VMEM((DC, D), BF16),
            pltpu.VMEM((DC, D), BF16),
            pltpu.VMEM((DC, D), BF16),
            pltpu.VMEM((S, DR), F32),
            pltpu.SemaphoreType.DMA((1 + 2 * NSTEP,)),
            pltpu.SemaphoreType.DMA((1 + 2 * NSTEP,)),
        ],
        compiler_params=pltpu.CompilerParams(
            collective_id=0,
            vmem_limit_bytes=64 * 1024 * 1024,
        ),
    )(x, Wdkv, Wuk, Wuv, Wq, Wqr, Wkr.T, Wo)
    return out
